# baseline (speedup 1.0000x reference)
import jax
import jax.numpy as jnp
from jax.experimental import pallas as pl
from jax.experimental.pallas import tpu as pltpu

_BM = 512
_NBUF = 6
_NSPLIT = 2


def _body(l_hbm, feat_hbm, w1_ref, w2_ref, b1_ref, b2_ref, o_ref,
          lbuf, feat_ref, sems, esem):
    i = pl.program_id(0)
    nchunk = pl.num_programs(0)
    n = l_hbm.shape[1]
    kh = n // _NSPLIT

    def start(chunk, buf):
        for s in range(_NSPLIT):
            pltpu.make_async_copy(
                l_hbm.at[pl.ds(chunk * _BM, _BM), pl.ds(s * kh, kh)],
                lbuf.at[buf, :, pl.ds(s * kh, kh)],
                sems.at[buf, s],
            ).start()

    @pl.when(i == 0)
    def _():
        pltpu.make_async_copy(feat_hbm, feat_ref, esem).start()
        for b in range(_NBUF):
            start(b, b)

    buf = jax.lax.rem(i, _NBUF)
    for s in range(_NSPLIT):
        pltpu.make_async_copy(
            l_hbm.at[pl.ds(i * _BM, _BM), pl.ds(s * kh, kh)],
            lbuf.at[buf, :, pl.ds(s * kh, kh)],
            sems.at[buf, s],
        ).wait()

    @pl.when(i == 0)
    def _():
        pltpu.make_async_copy(feat_hbm, feat_ref, esem).wait()

    x = jnp.dot(lbuf[buf], feat_ref[...], preferred_element_type=jnp.float32)
    e = feat_ref[pl.ds(i * _BM, _BM), :]
    dn = (((1,), (1,)), ((), ()))
    out = jax.lax.dot_general(e + x, w1_ref[...], dn,
                              preferred_element_type=jnp.float32)
    out += jax.lax.dot_general(x * e, w2_ref[...], dn,
                               preferred_element_type=jnp.float32)
    o_ref[...] = out + (b1_ref[...] + b2_ref[...])

    @pl.when(i + _NBUF < nchunk)
    def _():
        start(i + _NBUF, buf)


@jax.jit
def kernel(lap_matrix, eye_matrix, features, W1, b1, W2, b2):
    del eye_matrix
    n, d_in = features.shape
    d_out = W1.shape[0]
    grid = (n // _BM,)
    return pl.pallas_call(
        _body,
        grid=grid,
        in_specs=[
            pl.BlockSpec(memory_space=pl.ANY),
            pl.BlockSpec(memory_space=pl.ANY),
            pl.BlockSpec((d_out, d_in), lambda i: (0, 0)),
            pl.BlockSpec((d_out, d_in), lambda i: (0, 0)),
            pl.BlockSpec((1, d_out), lambda i: (0, 0)),
            pl.BlockSpec((1, d_out), lambda i: (0, 0)),
        ],
        out_specs=pl.BlockSpec((_BM, d_out), lambda i: (i, 0)),
        out_shape=jax.ShapeDtypeStruct((n, d_out), jnp.float32),
        scratch_shapes=[
            pltpu.VMEM((_NBUF, _BM, n), jnp.float32),
            pltpu.VMEM((n, d_in), jnp.float32),
            pltpu.SemaphoreType.DMA((_NBUF, _NSPLIT)),
            pltpu.SemaphoreType.DMA,
        ],
    )(lap_matrix, features, W1, W2,
      b1.reshape(1, d_out), b2.reshape(1, d_out))


# speedup vs baseline: 1.2114x; 1.2114x over previous
import jax
import jax.numpy as jnp
from jax.experimental import pallas as pl
from jax.experimental.pallas import tpu as pltpu

_BM = 512
_NBUF = 6
_NSPLIT = 2


def _body(l_hbm, feat_ref, w1_ref, w2_ref, b1_ref, b2_ref, o_ref, lbuf, sems):
    i = pl.program_id(0)
    nchunk = pl.num_programs(0)
    n = l_hbm.shape[1]
    kh = n // _NSPLIT

    def start(chunk, buf):
        for s in range(_NSPLIT):
            pltpu.make_async_copy(
                l_hbm.at[pl.ds(chunk * _BM, _BM), pl.ds(s * kh, kh)],
                lbuf.at[buf, :, pl.ds(s * kh, kh)],
                sems.at[buf, s],
            ).start()

    @pl.when(i == 0)
    def _():
        for b in range(_NBUF):
            start(b, b)

    buf = jax.lax.rem(i, _NBUF)
    for s in range(_NSPLIT):
        pltpu.make_async_copy(
            l_hbm.at[pl.ds(i * _BM, _BM), pl.ds(s * kh, kh)],
            lbuf.at[buf, :, pl.ds(s * kh, kh)],
            sems.at[buf, s],
        ).wait()

    x = jnp.dot(lbuf[buf], feat_ref[...], preferred_element_type=jnp.float32)
    e = feat_ref[pl.ds(i * _BM, _BM), :]
    dn = (((1,), (1,)), ((), ()))
    out = jax.lax.dot_general(e + x, w1_ref[...], dn,
                              preferred_element_type=jnp.float32)
    out += jax.lax.dot_general(x * e, w2_ref[...], dn,
                               preferred_element_type=jnp.float32)
    o_ref[...] = out + (b1_ref[...] + b2_ref[...])

    @pl.when(i + _NBUF < nchunk)
    def _():
        start(i + _NBUF, buf)


@jax.jit
def kernel(lap_matrix, eye_matrix, features, W1, b1, W2, b2):
    del eye_matrix
    n, d_in = features.shape
    d_out = W1.shape[0]
    grid = (n // _BM,)
    return pl.pallas_call(
        _body,
        grid=grid,
        in_specs=[
            pl.BlockSpec(memory_space=pl.ANY),
            pl.BlockSpec((n, d_in), lambda i: (0, 0)),
            pl.BlockSpec((d_out, d_in), lambda i: (0, 0)),
            pl.BlockSpec((d_out, d_in), lambda i: (0, 0)),
            pl.BlockSpec((1, d_out), lambda i: (0, 0)),
            pl.BlockSpec((1, d_out), lambda i: (0, 0)),
        ],
        out_specs=pl.BlockSpec((_BM, d_out), lambda i: (i, 0)),
        out_shape=jax.ShapeDtypeStruct((n, d_out), jnp.float32),
        scratch_shapes=[
            pltpu.VMEM((_NBUF, _BM, n), jnp.float32),
            pltpu.SemaphoreType.DMA((_NBUF, _NSPLIT)),
        ],
    )(lap_matrix, features, W1, W2,
      b1.reshape(1, d_out), b2.reshape(1, d_out))


# speedup vs baseline: 1.2208x; 1.0077x over previous
import jax
import jax.numpy as jnp
from jax.experimental import pallas as pl
from jax.experimental.pallas import tpu as pltpu

_BM = 256
_NBUF = 12
_NSPLIT = 2


def _body(l_hbm, feat_ref, w1_ref, w2_ref, b1_ref, b2_ref, o_ref, lbuf, sems):
    i = pl.program_id(0)
    nchunk = pl.num_programs(0)
    n = l_hbm.shape[1]
    kh = n // _NSPLIT

    def start(chunk, buf):
        for s in range(_NSPLIT):
            pltpu.make_async_copy(
                l_hbm.at[pl.ds(chunk * _BM, _BM), pl.ds(s * kh, kh)],
                lbuf.at[buf, :, pl.ds(s * kh, kh)],
                sems.at[buf, s],
            ).start()

    @pl.when(i == 0)
    def _():
        for b in range(_NBUF):
            start(b, b)

    buf = jax.lax.rem(i, _NBUF)
    for s in range(_NSPLIT):
        pltpu.make_async_copy(
            l_hbm.at[pl.ds(i * _BM, _BM), pl.ds(s * kh, kh)],
            lbuf.at[buf, :, pl.ds(s * kh, kh)],
            sems.at[buf, s],
        ).wait()

    x = jnp.dot(lbuf[buf], feat_ref[...], preferred_element_type=jnp.float32)
    e = feat_ref[pl.ds(i * _BM, _BM), :]
    dn = (((1,), (1,)), ((), ()))
    out = jax.lax.dot_general(e + x, w1_ref[...], dn,
                              preferred_element_type=jnp.float32)
    out += jax.lax.dot_general(x * e, w2_ref[...], dn,
                               preferred_element_type=jnp.float32)
    o_ref[...] = out + (b1_ref[...] + b2_ref[...])

    @pl.when(i + _NBUF < nchunk)
    def _():
        start(i + _NBUF, buf)


@jax.jit
def kernel(lap_matrix, eye_matrix, features, W1, b1, W2, b2):
    del eye_matrix
    n, d_in = features.shape
    d_out = W1.shape[0]
    grid = (n // _BM,)
    return pl.pallas_call(
        _body,
        grid=grid,
        in_specs=[
            pl.BlockSpec(memory_space=pl.ANY),
            pl.BlockSpec((n, d_in), lambda i: (0, 0)),
            pl.BlockSpec((d_out, d_in), lambda i: (0, 0)),
            pl.BlockSpec((d_out, d_in), lambda i: (0, 0)),
            pl.BlockSpec((1, d_out), lambda i: (0, 0)),
            pl.BlockSpec((1, d_out), lambda i: (0, 0)),
        ],
        out_specs=pl.BlockSpec((_BM, d_out), lambda i: (i, 0)),
        out_shape=jax.ShapeDtypeStruct((n, d_out), jnp.float32),
        scratch_shapes=[
            pltpu.VMEM((_NBUF, _BM, n), jnp.float32),
            pltpu.SemaphoreType.DMA((_NBUF, _NSPLIT)),
        ],
    )(lap_matrix, features, W1, W2,
      b1.reshape(1, d_out), b2.reshape(1, d_out))


# speedup vs baseline: 1.2288x; 1.0066x over previous
import jax
import jax.numpy as jnp
from jax.experimental import pallas as pl
from jax.experimental.pallas import tpu as pltpu

_BM = 256
_NBUF = 12
_NSPLIT = 1


def _body(l_hbm, feat_ref, w1_ref, w2_ref, b1_ref, b2_ref, o_ref, lbuf, sems):
    i = pl.program_id(0)
    nchunk = pl.num_programs(0)
    n = l_hbm.shape[1]
    kh = n // _NSPLIT

    def start(chunk, buf):
        for s in range(_NSPLIT):
            pltpu.make_async_copy(
                l_hbm.at[pl.ds(chunk * _BM, _BM), pl.ds(s * kh, kh)],
                lbuf.at[buf, :, pl.ds(s * kh, kh)],
                sems.at[buf, s],
            ).start()

    @pl.when(i == 0)
    def _():
        for b in range(_NBUF):
            start(b, b)

    buf = jax.lax.rem(i, _NBUF)
    for s in range(_NSPLIT):
        pltpu.make_async_copy(
            l_hbm.at[pl.ds(i * _BM, _BM), pl.ds(s * kh, kh)],
            lbuf.at[buf, :, pl.ds(s * kh, kh)],
            sems.at[buf, s],
        ).wait()

    x = jnp.dot(lbuf[buf], feat_ref[...], preferred_element_type=jnp.float32)
    e = feat_ref[pl.ds(i * _BM, _BM), :]
    dn = (((1,), (1,)), ((), ()))
    out = jax.lax.dot_general(e + x, w1_ref[...], dn,
                              preferred_element_type=jnp.float32)
    out += jax.lax.dot_general(x * e, w2_ref[...], dn,
                               preferred_element_type=jnp.float32)
    o_ref[...] = out + (b1_ref[...] + b2_ref[...])

    @pl.when(i + _NBUF < nchunk)
    def _():
        start(i + _NBUF, buf)


@jax.jit
def kernel(lap_matrix, eye_matrix, features, W1, b1, W2, b2):
    del eye_matrix
    n, d_in = features.shape
    d_out = W1.shape[0]
    grid = (n // _BM,)
    return pl.pallas_call(
        _body,
        grid=grid,
        in_specs=[
            pl.BlockSpec(memory_space=pl.ANY),
            pl.BlockSpec((n, d_in), lambda i: (0, 0)),
            pl.BlockSpec((d_out, d_in), lambda i: (0, 0)),
            pl.BlockSpec((d_out, d_in), lambda i: (0, 0)),
            pl.BlockSpec((1, d_out), lambda i: (0, 0)),
            pl.BlockSpec((1, d_out), lambda i: (0, 0)),
        ],
        out_specs=pl.BlockSpec((_BM, d_out), lambda i: (i, 0)),
        out_shape=jax.ShapeDtypeStruct((n, d_out), jnp.float32),
        scratch_shapes=[
            pltpu.VMEM((_NBUF, _BM, n), jnp.float32),
            pltpu.SemaphoreType.DMA((_NBUF, _NSPLIT)),
        ],
    )(lap_matrix, features, W1, W2,
      b1.reshape(1, d_out), b2.reshape(1, d_out))


# speedup vs baseline: 1.2340x; 1.0042x over previous
import jax
import jax.numpy as jnp
from jax.experimental import pallas as pl
from jax.experimental.pallas import tpu as pltpu

_BM = 256
_NBUF = 13
_NSPLIT = 1


def _body(l_hbm, feat_ref, w1_ref, w2_ref, b1_ref, b2_ref, o_ref, lbuf, sems):
    i = pl.program_id(0)
    nchunk = pl.num_programs(0)
    n = l_hbm.shape[1]
    kh = n // _NSPLIT

    def start(chunk, buf):
        for s in range(_NSPLIT):
            pltpu.make_async_copy(
                l_hbm.at[pl.ds(chunk * _BM, _BM), pl.ds(s * kh, kh)],
                lbuf.at[buf, :, pl.ds(s * kh, kh)],
                sems.at[buf, s],
            ).start()

    @pl.when(i == 0)
    def _():
        for b in range(_NBUF):
            start(b, b)

    buf = jax.lax.rem(i, _NBUF)
    for s in range(_NSPLIT):
        pltpu.make_async_copy(
            l_hbm.at[pl.ds(i * _BM, _BM), pl.ds(s * kh, kh)],
            lbuf.at[buf, :, pl.ds(s * kh, kh)],
            sems.at[buf, s],
        ).wait()

    x = jnp.dot(lbuf[buf], feat_ref[...], preferred_element_type=jnp.float32)
    e = feat_ref[pl.ds(i * _BM, _BM), :]
    dn = (((1,), (1,)), ((), ()))
    out = jax.lax.dot_general(e + x, w1_ref[...], dn,
                              preferred_element_type=jnp.float32)
    out += jax.lax.dot_general(x * e, w2_ref[...], dn,
                               preferred_element_type=jnp.float32)
    o_ref[...] = out + (b1_ref[...] + b2_ref[...])

    @pl.when(i + _NBUF < nchunk)
    def _():
        start(i + _NBUF, buf)


@jax.jit
def kernel(lap_matrix, eye_matrix, features, W1, b1, W2, b2):
    del eye_matrix
    n, d_in = features.shape
    d_out = W1.shape[0]
    grid = (n // _BM,)
    return pl.pallas_call(
        _body,
        grid=grid,
        in_specs=[
            pl.BlockSpec(memory_space=pl.ANY),
            pl.BlockSpec((n, d_in), lambda i: (0, 0)),
            pl.BlockSpec((d_out, d_in), lambda i: (0, 0)),
            pl.BlockSpec((d_out, d_in), lambda i: (0, 0)),
            pl.BlockSpec((1, d_out), lambda i: (0, 0)),
            pl.BlockSpec((1, d_out), lambda i: (0, 0)),
        ],
        out_specs=pl.BlockSpec((_BM, d_out), lambda i: (i, 0)),
        out_shape=jax.ShapeDtypeStruct((n, d_out), jnp.float32),
        scratch_shapes=[
            pltpu.VMEM((_NBUF, _BM, n), jnp.float32),
            pltpu.SemaphoreType.DMA((_NBUF, _NSPLIT)),
        ],
    )(lap_matrix, features, W1, W2,
      b1.reshape(1, d_out), b2.reshape(1, d_out))
